# baseline (speedup 1.0000x reference)
import dataclasses
import functools

import jax
import jax.numpy as jnp
from jax import lax
from jax.experimental import pallas as pl
from jax.experimental.pallas import tpu as pltpu
from jax.experimental.pallas import tpu_sc as plsc

N = 10000
D_H = 128
NC = 2
NS = 16
LANES = 16
NW = NC * NS
CB = 64
DEPTH = 3
NPAD = 10240
STRIPE = NPAD // NS
FRAC0 = 0.76

_SC_PARAMS = pltpu.CompilerParams()
if "needs_layout_passes" in pltpu.CompilerParams.__dataclass_fields__:
    _SC_PARAMS = dataclasses.replace(_SC_PARAMS, needs_layout_passes=False)


def _make_sc_prop(nca: int, ncb: int, d: int):
    grp = d // LANES
    ncmax = max(nca, ncb)
    mesh = plsc.VectorSubcoreMesh(core_axis_name="c", subcore_axis_name="s")

    @functools.partial(
        pl.kernel,
        out_type=jax.ShapeDtypeStruct((NC, NPAD, d), jnp.float32),
        mesh=mesh,
        scratch_types=(
            [pltpu.VMEM((ncmax * CB,), jnp.int32),
             pltpu.VMEM((DEPTH, CB), jnp.int32),
             pltpu.VMEM((DEPTH, CB), jnp.float32)]
            + [pltpu.VMEM((CB, d), jnp.float32)] * DEPTH
            + [pltpu.VMEM_SHARED((NPAD, d), jnp.float32)]
            + [pltpu.SemaphoreType.DMA] * (4 * DEPTH)
        ),
    )
    def prop(g_hbm, src_hbm, dst_hbm, w_hbm, out_hbm, src_v, dring, wring,
             *rest):
        rows = rest[:DEPTH]
        acc_sh = rest[DEPTH]
        sem_g = rest[DEPTH + 1:2 * DEPTH + 1]
        sem_s = rest[2 * DEPTH + 1:3 * DEPTH + 1]
        sem_d = rest[3 * DEPTH + 1:4 * DEPTH + 1]
        sem_w = rest[4 * DEPTH + 1:]
        cid = lax.axis_index("c")
        sid = lax.axis_index("s")
        nch = jnp.where(cid == 0, nca, ncb)
        ebase = jnp.where(cid == 0, sid * (nca * CB),
                          NS * (nca * CB) + sid * (ncb * CB))

        pltpu.sync_copy(src_hbm.at[pl.ds(ebase, ncmax * CB)], src_v)

        zeros = jnp.zeros((LANES,), jnp.float32)

        @pl.loop(0, CB)
        def _(r):
            for cg in range(grp):
                rows[0][r, pl.ds(cg * LANES, LANES)] = zeros

        for k in range(STRIPE // CB):
            pltpu.sync_copy(rows[0], acc_sh.at[pl.ds(sid * STRIPE + k * CB, CB)])
        plsc.subcore_barrier()

        def scale(buf, wq):
            @pl.loop(0, CB // LANES)
            def _(rg):
                wvec = wring[wq, pl.ds(rg * LANES, LANES)]
                for rr in range(LANES):
                    r = rg * LANES + rr
                    s = wvec[rr]
                    for cg in range(grp):
                        sl = pl.ds(cg * LANES, LANES)
                        buf[r, sl] = buf[r, sl] * s

        for q in range(DEPTH - 1):
            pltpu.async_copy(g_hbm.at[src_v.at[pl.ds(q * CB, CB)]],
                             rows[q], sem_g[q])
            pltpu.async_copy(dst_hbm.at[pl.ds(ebase + q * CB, CB)],
                             dring.at[q], sem_d[q])
            pltpu.async_copy(w_hbm.at[pl.ds(ebase + q * CB, CB)],
                             wring.at[q], sem_w[q])

        @pl.loop(0, nch, step=DEPTH)
        def _(j):
            for q in range(DEPTH):
                c = j + q
                bq = (q + DEPTH - 1) % DEPTH

                @pl.when(c + DEPTH - 1 < nch)
                def _():
                    @pl.when(c >= 1)
                    def _():
                        pltpu.make_async_copy(
                            rows[bq], acc_sh.at[dring.at[bq]],
                            sem_s[bq]).wait()
                    pltpu.async_copy(
                        g_hbm.at[src_v.at[pl.ds((c + DEPTH - 1) * CB, CB)]],
                        rows[bq], sem_g[bq])
                    pltpu.async_copy(
                        dst_hbm.at[pl.ds(ebase + (c + DEPTH - 1) * CB, CB)],
                        dring.at[bq], sem_d[bq])
                    pltpu.async_copy(
                        w_hbm.at[pl.ds(ebase + (c + DEPTH - 1) * CB, CB)],
                        wring.at[bq], sem_w[bq])

                pltpu.make_async_copy(g_hbm.at[src_v.at[pl.ds(c * CB, CB)]],
                                      rows[q], sem_g[q]).wait()
                pltpu.make_async_copy(dst_hbm.at[pl.ds(ebase + c * CB, CB)],
                                      dring.at[q], sem_d[q]).wait()
                pltpu.make_async_copy(w_hbm.at[pl.ds(ebase + c * CB, CB)],
                                      wring.at[q], sem_w[q]).wait()
                scale(rows[q], q)
                pltpu.async_copy(rows[q], acc_sh.at[dring.at[q]], sem_s[q],
                                 add=True)

        for q in range(DEPTH):
            pltpu.make_async_copy(rows[q], acc_sh.at[dring.at[q]],
                                  sem_s[q]).wait()

        plsc.subcore_barrier()
        for k in range(STRIPE // CB):
            sl = pl.ds(sid * STRIPE + k * CB, CB)
            pltpu.sync_copy(acc_sh.at[sl], out_hbm.at[cid, sl])

    return prop


def _make_sc_deg(nedge: int):
    mesh = plsc.VectorSubcoreMesh(core_axis_name="c", subcore_axis_name="s")

    @functools.partial(
        pl.kernel,
        out_type=jax.ShapeDtypeStruct((NW, NPAD), jnp.float32),
        mesh=mesh,
        compiler_params=_SC_PARAMS,
        scratch_types=[
            pltpu.VMEM((nedge,), jnp.int32),
            pltpu.VMEM((nedge,), jnp.float32),
            pltpu.VMEM((NPAD,), jnp.float32),
        ],
    )
    def deg(dst_hbm, w_hbm, out_hbm, dst_v, w_v, hist):
        cid = lax.axis_index("c")
        sid = lax.axis_index("s")
        wid = cid * NS + sid

        pltpu.sync_copy(dst_hbm.at[wid], dst_v)
        pltpu.sync_copy(w_hbm.at[wid], w_v)

        zeros = jnp.zeros((LANES,), jnp.float32)

        @pl.loop(0, NPAD, step=LANES)
        def _(i):
            hist[pl.ds(i, LANES)] = zeros

        @pl.loop(0, nedge, step=LANES)
        def _(e):
            sl = pl.ds(e, LANES)
            plsc.addupdate_scatter(hist, [dst_v[sl]], w_v[sl])

        pltpu.sync_copy(hist, out_hbm.at[wid])

    return deg


def _tc_prep(degp, x, W1):
    def body(deg_ref, x_ref, w_ref, dinv_ref, g_ref):
        degc = jnp.sum(deg_ref[...], axis=0)[:N, None] + 1.0
        dinv = lax.rsqrt(degc)
        dinv_ref[...] = dinv
        g_ref[...] = dinv * jnp.dot(x_ref[...], w_ref[...],
                                    preferred_element_type=jnp.float32)

    return pl.pallas_call(
        body,
        out_shape=[jax.ShapeDtypeStruct((N, 1), jnp.float32),
                   jax.ShapeDtypeStruct((N, D_H), jnp.float32)],
    )(degp, x, W1)


def _tc_mid(acc, g, dinv, b, Wn):
    dn = Wn.shape[1]

    def body(acc_ref, g_ref, dinv_ref, b_ref, w_ref, out_ref):
        a = acc_ref[0, :N, :] + acc_ref[1, :N, :] + g_ref[...]
        h = jnp.maximum(dinv_ref[...] * a + b_ref[...], 0.0)
        out_ref[...] = dinv_ref[...] * jnp.dot(h, w_ref[...],
                                               preferred_element_type=jnp.float32)

    return pl.pallas_call(
        body,
        out_shape=jax.ShapeDtypeStruct((N, dn), jnp.float32),
    )(acc, g, dinv, b.reshape(1, -1), Wn)


def _tc_mid_t(acc, g, dinv, b):
    def body(acc_ref, g_ref, dinv_ref, b_ref, out_ref):
        a = acc_ref[0, :N, :] + acc_ref[1, :N, :] + g_ref[...]
        h = jnp.maximum(dinv_ref[...] * a + b_ref[...], 0.0)
        out_ref[...] = dinv_ref[...] * h

    return pl.pallas_call(
        body,
        out_shape=jax.ShapeDtypeStruct((N, D_H), jnp.float32),
    )(acc, g, dinv, b.reshape(1, -1))


def _tc_fin(acc, t, dinv, W8, b8):
    def body(acc_ref, t_ref, dinv_ref, w_ref, b_ref, out_ref):
        a = acc_ref[0, :N, :] + acc_ref[1, :N, :] + t_ref[...]
        h = dinv_ref[...] * a
        out_ref[...] = jnp.dot(h, w_ref[...],
                               preferred_element_type=jnp.float32) + b_ref[...]

    return pl.pallas_call(
        body,
        out_shape=jax.ShapeDtypeStruct((N, 2), jnp.float32),
    )(acc, t, dinv, W8, b8.reshape(1, -1))


def kernel(x, edge_index, edge_attr, W1, b1, W2, b2, W3, b3, W4, b4, W5, b5,
           W8, b8):
    E = edge_index.shape[1]
    src = edge_index[0].astype(jnp.int32)
    dst = edge_index[1].astype(jnp.int32)
    w = edge_attr.astype(jnp.float32)

    gran = NS * CB * DEPTH
    sa = max(gran, int(round(E * FRAC0 / gran)) * gran)
    nca = sa // (NS * CB)
    sb = max(gran, -(-(E - sa) // gran) * gran)
    ncb = sb // (NS * CB)
    ltot = sa + sb + max(nca - ncb, 0) * CB
    srcf = jnp.pad(src, (0, ltot - E))
    dstf = jnp.pad(dst, (0, ltot - E))
    wf = jnp.pad(w, (0, ltot - E))

    prop128 = _make_sc_prop(nca, ncb, D_H)

    npt = -(-(-(-E // NW)) // LANES) * LANES
    du = jnp.pad(dst, (0, NW * npt - E)).reshape(NW, npt)
    wu = jnp.pad(w, (0, NW * npt - E)).reshape(NW, npt)
    degk = _make_sc_deg(npt)

    degp = degk(du, wu)
    dinv, g = _tc_prep(degp, x, W1)
    for b_l, W_next in ((b1, W2), (b2, W3), (b3, W4), (b4, W5)):
        acc = prop128(g, srcf, dstf, wf)
        g = _tc_mid(acc, g, dinv, b_l, W_next)

    acc = prop128(g, srcf, dstf, wf)
    t = _tc_mid_t(acc, g, dinv, b5)

    acc_t = prop128(t, srcf, dstf, wf)
    return _tc_fin(acc_t, t, dinv, W8, b8)


# speedup vs baseline: 1.0695x; 1.0695x over previous
import dataclasses
import functools

import jax
import jax.numpy as jnp
from jax import lax
from jax.experimental import pallas as pl
from jax.experimental.pallas import tpu as pltpu
from jax.experimental.pallas import tpu_sc as plsc

N = 10000
D_H = 128
NC = 2
NS = 16
LANES = 16
NW = NC * NS
CB = 64
DEPTH = 3
NPAD = 10240
STRIPE = NPAD // NS
FRAC0 = 0.73

_SC_PARAMS = pltpu.CompilerParams()
if "needs_layout_passes" in pltpu.CompilerParams.__dataclass_fields__:
    _SC_PARAMS = dataclasses.replace(_SC_PARAMS, needs_layout_passes=False)


def _make_sc_prop(nca: int, ncb: int, d: int):
    grp = d // LANES
    ncmax = max(nca, ncb)
    mesh = plsc.VectorSubcoreMesh(core_axis_name="c", subcore_axis_name="s")

    @functools.partial(
        pl.kernel,
        out_type=jax.ShapeDtypeStruct((NC, NPAD, d), jnp.float32),
        mesh=mesh,
        scratch_types=(
            [pltpu.VMEM((ncmax * CB,), jnp.int32),
             pltpu.VMEM((DEPTH, CB), jnp.int32),
             pltpu.VMEM((DEPTH, CB), jnp.float32)]
            + [pltpu.VMEM((CB, d), jnp.float32)] * DEPTH
            + [pltpu.VMEM_SHARED((NPAD, d), jnp.float32)]
            + [pltpu.SemaphoreType.DMA] * (4 * DEPTH)
        ),
    )
    def prop(g_hbm, src_hbm, dst_hbm, w_hbm, out_hbm, src_v, dring, wring,
             *rest):
        rows = rest[:DEPTH]
        acc_sh = rest[DEPTH]
        sem_g = rest[DEPTH + 1:2 * DEPTH + 1]
        sem_s = rest[2 * DEPTH + 1:3 * DEPTH + 1]
        sem_d = rest[3 * DEPTH + 1:4 * DEPTH + 1]
        sem_w = rest[4 * DEPTH + 1:]
        cid = lax.axis_index("c")
        sid = lax.axis_index("s")
        nch = jnp.where(cid == 0, nca, ncb)
        ebase = jnp.where(cid == 0, sid * (nca * CB),
                          NS * (nca * CB) + sid * (ncb * CB))

        pltpu.sync_copy(src_hbm.at[pl.ds(ebase, ncmax * CB)], src_v)

        zeros = jnp.zeros((LANES,), jnp.float32)

        @pl.loop(0, CB)
        def _(r):
            for cg in range(grp):
                rows[0][r, pl.ds(cg * LANES, LANES)] = zeros

        for k in range(STRIPE // CB):
            pltpu.sync_copy(rows[0], acc_sh.at[pl.ds(sid * STRIPE + k * CB, CB)])
        plsc.subcore_barrier()

        def scale(buf, wq):
            @pl.loop(0, CB // LANES)
            def _(rg):
                wvec = wring[wq, pl.ds(rg * LANES, LANES)]
                for rr in range(LANES):
                    r = rg * LANES + rr
                    s = wvec[rr]
                    for cg in range(grp):
                        sl = pl.ds(cg * LANES, LANES)
                        buf[r, sl] = buf[r, sl] * s

        for q in range(DEPTH - 1):
            pltpu.async_copy(g_hbm.at[src_v.at[pl.ds(q * CB, CB)]],
                             rows[q], sem_g[q])
            pltpu.async_copy(dst_hbm.at[pl.ds(ebase + q * CB, CB)],
                             dring.at[q], sem_d[q])
            pltpu.async_copy(w_hbm.at[pl.ds(ebase + q * CB, CB)],
                             wring.at[q], sem_w[q])

        @pl.loop(0, nch, step=DEPTH)
        def _(j):
            for q in range(DEPTH):
                c = j + q
                bq = (q + DEPTH - 1) % DEPTH

                @pl.when(c + DEPTH - 1 < nch)
                def _():
                    @pl.when(c >= 1)
                    def _():
                        pltpu.make_async_copy(
                            rows[bq], acc_sh.at[dring.at[bq]],
                            sem_s[bq]).wait()
                    pltpu.async_copy(
                        g_hbm.at[src_v.at[pl.ds((c + DEPTH - 1) * CB, CB)]],
                        rows[bq], sem_g[bq])
                    pltpu.async_copy(
                        dst_hbm.at[pl.ds(ebase + (c + DEPTH - 1) * CB, CB)],
                        dring.at[bq], sem_d[bq])
                    pltpu.async_copy(
                        w_hbm.at[pl.ds(ebase + (c + DEPTH - 1) * CB, CB)],
                        wring.at[bq], sem_w[bq])

                pltpu.make_async_copy(g_hbm.at[src_v.at[pl.ds(c * CB, CB)]],
                                      rows[q], sem_g[q]).wait()
                pltpu.make_async_copy(dst_hbm.at[pl.ds(ebase + c * CB, CB)],
                                      dring.at[q], sem_d[q]).wait()
                pltpu.make_async_copy(w_hbm.at[pl.ds(ebase + c * CB, CB)],
                                      wring.at[q], sem_w[q]).wait()
                scale(rows[q], q)
                pltpu.async_copy(rows[q], acc_sh.at[dring.at[q]], sem_s[q],
                                 add=True)

        for q in range(DEPTH):
            pltpu.make_async_copy(rows[q], acc_sh.at[dring.at[q]],
                                  sem_s[q]).wait()

        plsc.subcore_barrier()
        for k in range(STRIPE // CB):
            sl = pl.ds(sid * STRIPE + k * CB, CB)
            pltpu.sync_copy(acc_sh.at[sl], out_hbm.at[cid, sl])

    return prop


def _make_sc_deg(nedge: int):
    mesh = plsc.VectorSubcoreMesh(core_axis_name="c", subcore_axis_name="s")

    @functools.partial(
        pl.kernel,
        out_type=jax.ShapeDtypeStruct((NW, NPAD), jnp.float32),
        mesh=mesh,
        compiler_params=_SC_PARAMS,
        scratch_types=[
            pltpu.VMEM((nedge,), jnp.int32),
            pltpu.VMEM((nedge,), jnp.float32),
            pltpu.VMEM((NPAD,), jnp.float32),
        ],
    )
    def deg(dst_hbm, w_hbm, out_hbm, dst_v, w_v, hist):
        cid = lax.axis_index("c")
        sid = lax.axis_index("s")
        wid = cid * NS + sid

        pltpu.sync_copy(dst_hbm.at[wid], dst_v)
        pltpu.sync_copy(w_hbm.at[wid], w_v)

        zeros = jnp.zeros((LANES,), jnp.float32)

        @pl.loop(0, NPAD, step=LANES)
        def _(i):
            hist[pl.ds(i, LANES)] = zeros

        @pl.loop(0, nedge, step=LANES)
        def _(e):
            sl = pl.ds(e, LANES)
            plsc.addupdate_scatter(hist, [dst_v[sl]], w_v[sl])

        pltpu.sync_copy(hist, out_hbm.at[wid])

    return deg


def _tc_prep(degp, x, W1):
    def body(deg_ref, x_ref, w_ref, dinv_ref, g_ref):
        degc = jnp.sum(deg_ref[...], axis=0)[:N, None] + 1.0
        dinv = lax.rsqrt(degc)
        dinv_ref[...] = dinv
        g_ref[...] = dinv * jnp.dot(x_ref[...], w_ref[...],
                                    preferred_element_type=jnp.float32)

    return pl.pallas_call(
        body,
        out_shape=[jax.ShapeDtypeStruct((N, 1), jnp.float32),
                   jax.ShapeDtypeStruct((N, D_H), jnp.float32)],
    )(degp, x, W1)


def _tc_mid(acc, g, dinv, b, Wn):
    dn = Wn.shape[1]

    def body(acc_ref, g_ref, dinv_ref, b_ref, w_ref, out_ref):
        a = acc_ref[0, :N, :] + acc_ref[1, :N, :] + g_ref[...]
        h = jnp.maximum(dinv_ref[...] * a + b_ref[...], 0.0)
        out_ref[...] = dinv_ref[...] * jnp.dot(h, w_ref[...],
                                               preferred_element_type=jnp.float32)

    return pl.pallas_call(
        body,
        out_shape=jax.ShapeDtypeStruct((N, dn), jnp.float32),
    )(acc, g, dinv, b.reshape(1, -1), Wn)


def _tc_mid_t(acc, g, dinv, b):
    def body(acc_ref, g_ref, dinv_ref, b_ref, out_ref):
        a = acc_ref[0, :N, :] + acc_ref[1, :N, :] + g_ref[...]
        h = jnp.maximum(dinv_ref[...] * a + b_ref[...], 0.0)
        out_ref[...] = dinv_ref[...] * h

    return pl.pallas_call(
        body,
        out_shape=jax.ShapeDtypeStruct((N, D_H), jnp.float32),
    )(acc, g, dinv, b.reshape(1, -1))


def _tc_fin(acc, t, dinv, W8, b8):
    def body(acc_ref, t_ref, dinv_ref, w_ref, b_ref, out_ref):
        a = acc_ref[0, :N, :] + acc_ref[1, :N, :] + t_ref[...]
        h = dinv_ref[...] * a
        out_ref[...] = jnp.dot(h, w_ref[...],
                               preferred_element_type=jnp.float32) + b_ref[...]

    return pl.pallas_call(
        body,
        out_shape=jax.ShapeDtypeStruct((N, 2), jnp.float32),
    )(acc, t, dinv, W8, b8.reshape(1, -1))


def kernel(x, edge_index, edge_attr, W1, b1, W2, b2, W3, b3, W4, b4, W5, b5,
           W8, b8):
    E = edge_index.shape[1]
    src = edge_index[0].astype(jnp.int32)
    dst = edge_index[1].astype(jnp.int32)
    w = edge_attr.astype(jnp.float32)

    gran = NS * CB * DEPTH
    sa = max(gran, int(round(E * FRAC0 / gran)) * gran)
    nca = sa // (NS * CB)
    sb = max(gran, -(-(E - sa) // gran) * gran)
    ncb = sb // (NS * CB)
    ltot = sa + sb + max(nca - ncb, 0) * CB
    srcf = jnp.pad(src, (0, ltot - E))
    dstf = jnp.pad(dst, (0, ltot - E))
    wf = jnp.pad(w, (0, ltot - E))

    prop128 = _make_sc_prop(nca, ncb, D_H)

    npt = -(-(-(-E // NW)) // LANES) * LANES
    du = jnp.pad(dst, (0, NW * npt - E)).reshape(NW, npt)
    wu = jnp.pad(w, (0, NW * npt - E)).reshape(NW, npt)
    degk = _make_sc_deg(npt)

    degp = degk(du, wu)
    dinv, g = _tc_prep(degp, x, W1)
    for b_l, W_next in ((b1, W2), (b2, W3), (b3, W4), (b4, W5)):
        acc = prop128(g, srcf, dstf, wf)
        g = _tc_mid(acc, g, dinv, b_l, W_next)

    acc = prop128(g, srcf, dstf, wf)
    t = _tc_mid_t(acc, g, dinv, b5)

    acc_t = prop128(t, srcf, dstf, wf)
    return _tc_fin(acc_t, t, dinv, W8, b8)
